# baseline (speedup 1.0000x reference)
import functools

import jax
import jax.numpy as jnp
from jax.experimental import pallas as pl
from jax.experimental.pallas import tpu as pltpu

HIDDEN = 1024
HEADS = 16
DH = 64
SEG = 128
SCALE = DH ** -0.5


def _fused_kernel(x_ref, wqvT_ref, wk_ref, woutT_ref, eb_ref, out_ref,
                  kbig_ref, vbig_ref):
    b = pl.program_id(0)
    s = pl.program_id(1)

    @pl.when((b == 0) & (s == 0))
    def _init():
        kbig_ref[...] = jnp.zeros_like(kbig_ref)
        vbig_ref[...] = jnp.zeros_like(vbig_ref)

    xb = x_ref[0].astype(jnp.bfloat16)
    qv = jnp.dot(xb, wqvT_ref[...], preferred_element_type=jnp.float32)
    q = (qv[:, :HIDDEN] * SCALE).astype(jnp.bfloat16)
    v = qv[:, HIDDEN:].astype(jnp.bfloat16)
    kT = jax.lax.dot_general(wk_ref[...], xb, (((1,), (1,)), ((), ())),
                             preferred_element_type=jnp.float32
                             ).astype(jnp.bfloat16)

    ii = jax.lax.broadcasted_iota(jnp.int32, (64, 64), 0)
    jj = jax.lax.broadcasted_iota(jnp.int32, (64, 64), 1)
    perm = jnp.where(ii + jj == 63, 1.0, 0.0).astype(jnp.bfloat16)
    q_flip = jnp.dot(perm, q[64:, :],
                     preferred_element_type=jnp.float32).astype(jnp.bfloat16)
    q2 = jnp.concatenate([q[:64, :], q_flip], axis=0)

    for h in range(HEADS):
        kbig_ref[h * DH:(h + 1) * DH, h * SEG:(h + 1) * SEG] = \
            kT[h * DH:(h + 1) * DH, :]
        vbig_ref[h * SEG:(h + 1) * SEG, h * DH:(h + 1) * DH] = \
            v[:, h * DH:(h + 1) * DH]

    S = jnp.dot(q2, kbig_ref[...],
                preferred_element_type=jnp.float32)
    c0 = jnp.max(S[:64, :], axis=0, keepdims=True)
    c1 = jnp.max(S[64:, :], axis=0, keepdims=True)
    W = jnp.exp(S - jnp.concatenate(
        [jnp.broadcast_to(c0, (64, HEADS * SEG)),
         jnp.broadcast_to(c1, (64, HEADS * SEG))], axis=0))
    Wb = W.astype(jnp.bfloat16)
    num = jnp.dot(Wb, vbig_ref[...],
                  preferred_element_type=jnp.float32)
    den = jnp.dot(Wb, eb_ref[...],
                  preferred_element_type=jnp.float32)
    attn = (num / (1e-10 + den)).astype(jnp.bfloat16)

    out_ref[0] = jnp.dot(attn, woutT_ref[...],
                         preferred_element_type=jnp.float32)


@functools.partial(jax.jit, static_argnums=())
def kernel(x, Wqkv, Wout):
    B, M, D = x.shape
    nseg = M // SEG
    wqkvT = Wqkv.T.astype(jnp.bfloat16)
    wqvT = jnp.concatenate([wqkvT[:, :HIDDEN], wqkvT[:, 2 * HIDDEN:]], axis=1)
    wk = Wqkv[HIDDEN:2 * HIDDEN, :].astype(jnp.bfloat16)
    woutT = Wout.T.astype(jnp.bfloat16)
    jrow = jnp.arange(HEADS * SEG)[:, None] // SEG
    ccol = jnp.arange(HIDDEN)[None, :] // DH
    eb = (jrow == ccol).astype(jnp.bfloat16)

    return pl.pallas_call(
        _fused_kernel,
        grid=(B, nseg),
        in_specs=[
            pl.BlockSpec((1, SEG, HIDDEN), lambda b, s: (b, s, 0)),
            pl.BlockSpec((HIDDEN, 2 * HIDDEN), lambda b, s: (0, 0)),
            pl.BlockSpec((HIDDEN, HIDDEN), lambda b, s: (0, 0)),
            pl.BlockSpec((HIDDEN, HIDDEN), lambda b, s: (0, 0)),
            pl.BlockSpec((HEADS * SEG, HIDDEN), lambda b, s: (0, 0)),
        ],
        out_specs=pl.BlockSpec((1, SEG, HIDDEN), lambda b, s: (b, s, 0)),
        out_shape=jax.ShapeDtypeStruct((B, M, D), jnp.float32),
        scratch_shapes=[
            pltpu.VMEM((HIDDEN, HEADS * SEG), jnp.bfloat16),
            pltpu.VMEM((HEADS * SEG, HIDDEN), jnp.bfloat16),
        ],
        compiler_params=pltpu.CompilerParams(
            dimension_semantics=("arbitrary", "arbitrary")),
    )(x, wqvT, wk, woutT, eb)


# speedup vs baseline: 1.4946x; 1.4946x over previous
import functools

import jax
import jax.numpy as jnp
from jax.experimental import pallas as pl
from jax.experimental.pallas import tpu as pltpu

HIDDEN = 1024
HEADS = 16
DH = 64
SEG = 128
SCALE = DH ** -0.5
NKEY = HEADS * SEG
VAUG = HIDDEN + SEG


def _fused_kernel(x_ref, wqkvT_ref, woutT_ref, out_ref, vaug_ref):
    b = pl.program_id(0)
    s = pl.program_id(1)

    @pl.when((b == 0) & (s == 0))
    def _init():
        rr = jax.lax.broadcasted_iota(jnp.int32, (NKEY, VAUG), 0)
        cc = jax.lax.broadcasted_iota(jnp.int32, (NKEY, VAUG), 1)
        vaug_ref[...] = (cc == HIDDEN + rr // SEG).astype(jnp.bfloat16)

    xb = x_ref[0].astype(jnp.bfloat16)
    qkv = jnp.dot(xb, wqkvT_ref[...], preferred_element_type=jnp.float32)
    q = (qkv[:, :HIDDEN] * SCALE).astype(jnp.bfloat16)
    k = qkv[:, HIDDEN:2 * HIDDEN].astype(jnp.bfloat16)
    v = qkv[:, 2 * HIDDEN:].astype(jnp.bfloat16)

    ii = jax.lax.broadcasted_iota(jnp.int32, (64, 64), 0)
    jj = jax.lax.broadcasted_iota(jnp.int32, (64, 64), 1)
    perm = jnp.where(ii + jj == 63, 1.0, 0.0).astype(jnp.bfloat16)
    q_flip = jnp.dot(perm, q[64:, :],
                     preferred_element_type=jnp.float32).astype(jnp.bfloat16)
    q2 = jnp.concatenate([q[:64, :], q_flip], axis=0)

    parts = []
    for h in range(HEADS):
        sl = slice(h * DH, (h + 1) * DH)
        parts.append(jax.lax.dot_general(
            q2[:, sl], k[:, sl], (((1,), (1,)), ((), ())),
            preferred_element_type=jnp.float32))
        vaug_ref[h * SEG:(h + 1) * SEG, sl] = v[:, sl]
    S = jnp.concatenate(parts, axis=1)

    c0 = jnp.max(S[:64, :], axis=0, keepdims=True)
    c1 = jnp.max(S[64:, :], axis=0, keepdims=True)
    W = jnp.exp(S - jnp.concatenate([jnp.broadcast_to(c0, (64, NKEY)),
                                     jnp.broadcast_to(c1, (64, NKEY))],
                                    axis=0))
    Wb = W.astype(jnp.bfloat16)

    nd = jnp.dot(Wb, vaug_ref[...],
                 preferred_element_type=jnp.float32)
    num = nd[:, :HIDDEN]
    den_small = nd[:, HIDDEN:HIDDEN + HEADS].astype(jnp.bfloat16)
    hh = jax.lax.broadcasted_iota(jnp.int32, (HEADS, HIDDEN), 0)
    cc2 = jax.lax.broadcasted_iota(jnp.int32, (HEADS, HIDDEN), 1)
    bmap = (hh == cc2 // DH).astype(jnp.bfloat16)
    den = jnp.dot(den_small, bmap,
                  preferred_element_type=jnp.float32)
    attn = (num / (1e-10 + den)).astype(jnp.bfloat16)

    out_ref[0] = jnp.dot(attn, woutT_ref[...],
                         preferred_element_type=jnp.float32)


@functools.partial(jax.jit, static_argnums=())
def kernel(x, Wqkv, Wout):
    B, M, D = x.shape
    nseg = M // SEG
    wqkvT = Wqkv.T.astype(jnp.bfloat16)
    woutT = Wout.T.astype(jnp.bfloat16)

    return pl.pallas_call(
        _fused_kernel,
        grid=(B, nseg),
        in_specs=[
            pl.BlockSpec((1, SEG, HIDDEN), lambda b, s: (b, s, 0)),
            pl.BlockSpec((HIDDEN, 3 * HIDDEN), lambda b, s: (0, 0)),
            pl.BlockSpec((HIDDEN, HIDDEN), lambda b, s: (0, 0)),
        ],
        out_specs=pl.BlockSpec((1, SEG, HIDDEN), lambda b, s: (b, s, 0)),
        out_shape=jax.ShapeDtypeStruct((B, M, D), jnp.float32),
        scratch_shapes=[pltpu.VMEM((NKEY, VAUG), jnp.bfloat16)],
        compiler_params=pltpu.CompilerParams(
            dimension_semantics=("arbitrary", "arbitrary")),
    )(x, wqkvT, woutT)


# speedup vs baseline: 1.6353x; 1.0941x over previous
import functools

import jax
import jax.numpy as jnp
from jax.experimental import pallas as pl
from jax.experimental.pallas import tpu as pltpu

HIDDEN = 1024
HEADS = 16
DH = 64
SEG = 128
SCALE = DH ** -0.5
NKEY = HEADS * SEG
VAUG = HIDDEN + SEG
SEGS_PER_BLK = 2
BLK = SEG * SEGS_PER_BLK


def _fused_kernel(x_ref, wqkvT_ref, woutT_ref, perm_ref, bmap_ref, out_ref,
                  vaug_ref):
    b = pl.program_id(0)
    s = pl.program_id(1)

    @pl.when((b == 0) & (s == 0))
    def _init():
        rr = jax.lax.broadcasted_iota(jnp.int32, (NKEY, VAUG), 0)
        cc = jax.lax.broadcasted_iota(jnp.int32, (NKEY, VAUG), 1)
        ini = (cc == HIDDEN + rr // SEG).astype(jnp.bfloat16)
        for g in range(SEGS_PER_BLK):
            vaug_ref[g] = ini

    xb = x_ref[0].astype(jnp.bfloat16)
    qkv = jnp.dot(xb, wqkvT_ref[...], preferred_element_type=jnp.float32)
    q = (qkv[:, :HIDDEN] * SCALE).astype(jnp.bfloat16)
    k = qkv[:, HIDDEN:2 * HIDDEN].astype(jnp.bfloat16)
    v = qkv[:, 2 * HIDDEN:].astype(jnp.bfloat16)

    attn_parts = []
    for g in range(SEGS_PER_BLK):
        r0 = g * SEG
        q_flip = jnp.dot(perm_ref[...], q[r0 + 64:r0 + SEG, :],
                         preferred_element_type=jnp.float32
                         ).astype(jnp.bfloat16)
        q2 = jnp.concatenate([q[r0:r0 + 64, :], q_flip], axis=0)

        parts = []
        for h in range(HEADS):
            sl = slice(h * DH, (h + 1) * DH)
            parts.append(jax.lax.dot_general(
                q2[:, sl], k[r0:r0 + SEG, sl], (((1,), (1,)), ((), ())),
                preferred_element_type=jnp.float32))
            vaug_ref[g, h * SEG:(h + 1) * SEG, sl] = v[r0:r0 + SEG, sl]
        S = jnp.concatenate(parts, axis=1)

        c0 = jnp.max(S[:64, :], axis=0, keepdims=True)
        c1 = jnp.max(S[64:, :], axis=0, keepdims=True)
        W = jnp.exp(S - jnp.concatenate([jnp.broadcast_to(c0, (64, NKEY)),
                                         jnp.broadcast_to(c1, (64, NKEY))],
                                        axis=0))
        Wb = W.astype(jnp.bfloat16)

        nd = jnp.dot(Wb, vaug_ref[g],
                     preferred_element_type=jnp.float32)
        num = nd[:, :HIDDEN]
        den_small = nd[:, HIDDEN:HIDDEN + HEADS].astype(jnp.bfloat16)
        den = jnp.dot(den_small, bmap_ref[...],
                      preferred_element_type=jnp.float32)
        attn_parts.append((num / (1e-10 + den)).astype(jnp.bfloat16))

    attn = jnp.concatenate(attn_parts, axis=0)
    out_ref[0] = jnp.dot(attn, woutT_ref[...],
                         preferred_element_type=jnp.float32)


@functools.partial(jax.jit, static_argnums=())
def kernel(x, Wqkv, Wout):
    B, M, D = x.shape
    nblk = M // BLK
    wqkvT = Wqkv.T.astype(jnp.bfloat16)
    woutT = Wout.T.astype(jnp.bfloat16)
    i64 = jnp.arange(64)
    perm = (i64[:, None] + i64[None, :] == 63).astype(jnp.bfloat16)
    hh = jnp.arange(HEADS)[:, None]
    cc = jnp.arange(HIDDEN)[None, :] // DH
    bmap = (hh == cc).astype(jnp.bfloat16)

    return pl.pallas_call(
        _fused_kernel,
        grid=(B, nblk),
        in_specs=[
            pl.BlockSpec((1, BLK, HIDDEN), lambda b, s: (b, s, 0)),
            pl.BlockSpec((HIDDEN, 3 * HIDDEN), lambda b, s: (0, 0)),
            pl.BlockSpec((HIDDEN, HIDDEN), lambda b, s: (0, 0)),
            pl.BlockSpec((64, 64), lambda b, s: (0, 0)),
            pl.BlockSpec((HEADS, HIDDEN), lambda b, s: (0, 0)),
        ],
        out_specs=pl.BlockSpec((1, BLK, HIDDEN), lambda b, s: (b, s, 0)),
        out_shape=jax.ShapeDtypeStruct((B, M, D), jnp.float32),
        scratch_shapes=[pltpu.VMEM((SEGS_PER_BLK, NKEY, VAUG), jnp.bfloat16)],
        compiler_params=pltpu.CompilerParams(
            dimension_semantics=("arbitrary", "arbitrary")),
    )(x, wqkvT, woutT, perm, bmap)
